# baseline (speedup 1.0000x reference)
import functools

import jax
import jax.numpy as jnp
from jax import lax
from jax.experimental import pallas as pl
from jax.experimental.pallas import tpu as pltpu
from jax.experimental.pallas import tpu_sc as plsc

B = 16384
F = 64
NC = 2
NS = 16
NW = NC * NS
BPW = B // NW


@functools.lru_cache(maxsize=1)
def _make_gather3():
    mesh = plsc.VectorSubcoreMesh(core_axis_name="c", subcore_axis_name="s")

    @functools.partial(
        pl.kernel,
        mesh=mesh,
        out_type=[
            jax.ShapeDtypeStruct((B, 2 * F), jnp.float32),
            jax.ShapeDtypeStruct((B, 2 * F), jnp.float32),
            jax.ShapeDtypeStruct((B, 2 * F), jnp.float32),
        ],
        scratch_types=[
            pltpu.VMEM((BPW,), jnp.int32),
            pltpu.VMEM((BPW, 2 * F), jnp.float32),
            pltpu.SemaphoreType.DMA,
        ],
    )
    def _gather3(ut2, mt2, gt2, uidx, midx, gidx, out_u, out_m, out_g,
                 iv, rv, sem):
        wid = lax.axis_index("s") * NC + lax.axis_index("c")
        base = wid * BPW

        def one_table(table, idx, out):
            pltpu.sync_copy(idx.at[pl.ds(base, BPW)], iv)
            pltpu.async_copy(table.at[iv], rv, sem).wait()
            pltpu.sync_copy(rv, out.at[pl.ds(base, BPW)])

        one_table(ut2, uidx, out_u)
        one_table(mt2, midx, out_m)
        one_table(gt2, gidx, out_g)

    return _gather3


BT = 2048
GRID = B // BT


def _half(pairs, par):
    p = par[0, 0, :].astype(jnp.float32)[:, None]
    return pairs[:, :F] * (1.0 - p) + pairs[:, F:] * p


def _mlp_body(ur, mr, gr, up, mp, gp, w1u, w1m, w1g, b1, w2, b2, w3, b3,
              out):
    ue = _half(ur[...], up)
    me = _half(mr[...], mp)
    ge = _half(gr[...], gp)
    x = (jnp.dot(ue, w1u[...], preferred_element_type=jnp.float32)
         + jnp.dot(me, w1m[...], preferred_element_type=jnp.float32)
         + jnp.dot(ge, w1g[...], preferred_element_type=jnp.float32)
         + b1[...])
    x = jnp.maximum(x, 0.0)
    x = jnp.maximum(
        jnp.dot(x, w2[...], preferred_element_type=jnp.float32) + b2[...], 0.0)
    out[...] = jnp.dot(x, w3[...], preferred_element_type=jnp.float32) + b3[...]


def _mlp(ur, mr, gr, up, mp, gp, w1u, w1m, w1g, b1, w2, b2, w3, b3,
         *, interpret=False):
    full = lambda shape: pl.BlockSpec(shape, lambda i: (0, 0))
    row_spec = pl.BlockSpec((BT, 2 * F), lambda i: (i, 0))
    par_spec = pl.BlockSpec((1, 1, BT), lambda i: (i, 0, 0))
    return pl.pallas_call(
        _mlp_body,
        grid=(GRID,),
        in_specs=[
            row_spec, row_spec, row_spec,
            par_spec, par_spec, par_spec,
            full((F, 128)),
            full((F, 128)),
            full((F, 128)),
            full((1, 128)),
            full((128, F)),
            full((1, F)),
            full((F, 1)),
            full((1, 1)),
        ],
        out_specs=pl.BlockSpec((BT, 1), lambda i: (i, 0)),
        out_shape=jax.ShapeDtypeStruct((B, 1), jnp.float32),
        interpret=interpret,
    )(ur, mr, gr, up, mp, gp, w1u, w1m, w1g, b1, w2, b2, w3, b3)


def kernel(user, movie, genres, user_table, movie_table, genre_table,
           W1, b1, W2, b2, W3, b3):
    ur, mr, gr = _make_gather3()(user_table.reshape(-1, 2 * F),
                                 movie_table.reshape(-1, 2 * F),
                                 genre_table.reshape(-1, 2 * F),
                                 user >> 1, movie >> 1, genres >> 1)
    up = (user & 1).reshape(GRID, 1, BT)
    mp = (movie & 1).reshape(GRID, 1, BT)
    gp = (genres & 1).reshape(GRID, 1, BT)
    w1u = W1[:, :F].T
    w1m = W1[:, F:2 * F].T
    w1g = W1[:, 2 * F:].T
    return _mlp(ur, mr, gr, up, mp, gp, w1u, w1m, w1g,
                b1.reshape(1, 128), W2.T, b2.reshape(1, F),
                W3.T, b3.reshape(1, 1))


# speedup vs baseline: 1.5634x; 1.5634x over previous
import functools

import jax
import jax.numpy as jnp
from jax import lax
from jax.experimental import pallas as pl
from jax.experimental.pallas import tpu as pltpu
from jax.experimental.pallas import tpu_sc as plsc

B = 16384
F = 64
NC = 2
NS = 16
NW = NC * NS
BPW = B // NW
WIN = 16


@functools.lru_cache(maxsize=2)
def _make_gather(n_tables):
    mesh = plsc.VectorSubcoreMesh(core_axis_name="c", subcore_axis_name="s")

    @functools.partial(
        pl.kernel,
        mesh=mesh,
        out_type=[jax.ShapeDtypeStruct((B, F), jnp.float32)] * n_tables,
        scratch_types=[
            pltpu.VMEM((BPW + 16,), jnp.int32),
            pltpu.VMEM((BPW, F), jnp.float32),
            pltpu.SemaphoreType.DMA,
        ],
    )
    def _gather(*args):
        tables = args[:n_tables]
        idxs = args[n_tables:2 * n_tables]
        outs = args[2 * n_tables:3 * n_tables]
        iv, rows, sem = args[3 * n_tables:]
        wid = lax.axis_index("s") * NC + lax.axis_index("c")
        base = wid * BPW

        def one_table(table, idx, out):
            pltpu.sync_copy(idx.at[pl.ds(base, BPW)], iv.at[pl.ds(0, BPW)])

            def step(r, _):
                s = iv[pl.ds(r, 16)][0]
                pltpu.async_copy(
                    table.at[pl.ds(s, 1)], rows.at[pl.ds(r, 1)], sem)

                @pl.when(r >= WIN)
                def _():
                    pltpu.make_async_copy(
                        table.at[pl.ds(0, 1)], rows.at[pl.ds(0, 1)],
                        sem).wait()

                return 0

            lax.fori_loop(0, BPW, step, 0)
            for _ in range(WIN):
                pltpu.make_async_copy(
                    table.at[pl.ds(0, 1)], rows.at[pl.ds(0, 1)], sem).wait()
            pltpu.sync_copy(rows, out.at[pl.ds(base, BPW)])

        for t, i, o in zip(tables, idxs, outs):
            one_table(t, i, o)

    return _gather


BT = 2048
GRID = B // BT


def _mlp_body(ue, me, ge, w1u, w1m, w1g, b1, w2, b2, w3, b3, out):
    x = (jnp.dot(ue[...], w1u[...], preferred_element_type=jnp.float32)
         + jnp.dot(me[...], w1m[...], preferred_element_type=jnp.float32)
         + jnp.dot(ge[...], w1g[...], preferred_element_type=jnp.float32)
         + b1[...])
    x = jnp.maximum(x, 0.0)
    x = jnp.maximum(
        jnp.dot(x, w2[...], preferred_element_type=jnp.float32) + b2[...], 0.0)
    out[...] = jnp.dot(x, w3[...], preferred_element_type=jnp.float32) + b3[...]


def _mlp(ue, me, ge, w1u, w1m, w1g, b1, w2, b2, w3, b3, *, interpret=False):
    full = lambda shape: pl.BlockSpec(shape, lambda i: (0, 0))
    return pl.pallas_call(
        _mlp_body,
        grid=(GRID,),
        in_specs=[
            pl.BlockSpec((BT, F), lambda i: (i, 0)),
            pl.BlockSpec((BT, F), lambda i: (i, 0)),
            pl.BlockSpec((BT, F), lambda i: (i, 0)),
            full((F, 128)),
            full((F, 128)),
            full((F, 128)),
            full((1, 128)),
            full((128, F)),
            full((1, F)),
            full((F, 1)),
            full((1, 1)),
        ],
        out_specs=pl.BlockSpec((BT, 1), lambda i: (i, 0)),
        out_shape=jax.ShapeDtypeStruct((B, 1), jnp.float32),
        interpret=interpret,
    )(ue, me, ge, w1u, w1m, w1g, b1, w2, b2, w3, b3)


def kernel(user, movie, genres, user_table, movie_table, genre_table,
           W1, b1, W2, b2, W3, b3):
    me, ge = _make_gather(2)(movie_table, genre_table, movie, genres)
    ue, = _make_gather(1)(user_table, user)
    w1u = W1[:, :F].T
    w1m = W1[:, F:2 * F].T
    w1g = W1[:, 2 * F:].T
    return _mlp(ue, me, ge, w1u, w1m, w1g,
                b1.reshape(1, 128), W2.T, b2.reshape(1, F),
                W3.T, b3.reshape(1, 1))
